# baseline (speedup 1.0000x reference)
import jax
import jax.numpy as jnp
from jax.experimental import pallas as pl
from jax.experimental.pallas import tpu as pltpu
from jax.experimental.pallas import tpu_sc as plsc

_LAGS = (1, 5, 21)
_K = 8
_B, _N, _T, _H = 2, 32, 512, 256
_L = len(_LAGS)
_CPAD = 96
_NEG_INF = float("-inf")
_SCALE = 0.0625


def _bf16_round(x):
    return x.astype(jnp.bfloat16).astype(jnp.float32)


_NC = 4
_NCH = _N // _NC


def _select_body(peer_ref, tgt_ref, wq_ref, bq_ref, wk_ref, bk_ref,
                 w_ref, idx_ref, qk1_ref, qk5_ref, qk21_ref, kb_ref, logits_ref):
    b = pl.program_id(0)
    c = pl.program_id(1)

    @pl.when(c == 0)
    def _():
        qT = jax.lax.dot_general(wq_ref[...], tgt_ref[0], (((1,), (1,)), ((), ())),
                                 preferred_element_type=jnp.float32)
        qT = ((qT + jnp.transpose(bq_ref[...])).astype(jnp.bfloat16)
              .astype(jnp.float32) * _SCALE)
        for lag, qref in ((1, qk1_ref), (5, qk5_ref), (21, qk21_ref)):
            qref[:, : _T - lag] = qT[:, lag:]
            qref[:, _T - lag:] = jnp.zeros((_H, lag), jnp.float32)

    kT = jax.lax.dot_general(
        wk_ref[...], peer_ref[0].reshape(_NCH * _T, _H), (((1,), (1,)), ((), ())),
        preferred_element_type=jnp.float32)
    kb_ref[...] = (kT + jnp.transpose(bk_ref[...])).astype(jnp.bfloat16).astype(jnp.float32)

    def n_body(i, carry):
        for j in (i, i + _NCH // 2):
            kb = kb_ref[:, pl.ds(j * _T, _T)]
            n = c * _NCH + j
            for l, (lag, qref) in enumerate(((1, qk1_ref), (5, qk5_ref), (21, qk21_ref))):
                d = jnp.sum(qref[...] * kb, axis=0, keepdims=True)
                logits_ref[pl.ds(l * _N + n, 1), pl.ds(lag, _T - lag)] = d[:, : _T - lag]
        return carry

    jax.lax.fori_loop(0, _NCH // 2, n_body, 0)

    @pl.when(c == _NC - 1)
    def _():
        _finish(b, w_ref, idx_ref, logits_ref)


def _finish(b, w_ref, idx_ref, logits_ref):
    for l, lag in enumerate(_LAGS):
        logits_ref[pl.ds(l * _N, _N), pl.ds(0, lag)] = jnp.full((_N, lag), _NEG_INF,
                                                                jnp.float32)

    arr = logits_ref[...]
    rows = jax.lax.broadcasted_iota(jnp.int32, (_CPAD, _T), 0)
    vals, idxs = [], []
    for _ in range(_K):
        mmax = jnp.max(arr, axis=0, keepdims=True)
        sel = jnp.where(arr == mmax, rows, _CPAD)
        amin = jnp.min(sel, axis=0, keepdims=True)
        vals.append(mmax)
        idxs.append(amin)
        arr = jnp.where(rows == amin, _NEG_INF, arr)
    vals8 = jnp.concatenate(vals, axis=0)
    idx8 = jnp.concatenate(idxs, axis=0)

    allinf = vals8[0:1, :] == _NEG_INF
    safe = jnp.maximum(vals8, -1e9)
    e = jnp.exp(safe - safe[0:1, :])
    w = e / jnp.sum(e, axis=0, keepdims=True)
    w = jnp.where(allinf, 0.0, w)

    l_id = jax.lax.shift_right_logical(idx8, 5)
    n_id = jnp.bitwise_and(idx8, 31)
    lag_v = jnp.where(l_id == 0, 1, jnp.where(l_id == 1, 5, 21))
    t_iota = jax.lax.broadcasted_iota(jnp.int32, (_K, _T), 1)
    tt = jnp.maximum(t_iota - lag_v, 0)
    w_ref[0] = w
    idx_ref[0] = (b * _N + n_id) * _T + tt


def _select(peer_t, tgt_t, Wq, bq_row, Wk, bk_row):
    return pl.pallas_call(
        _select_body,
        grid=(_B, _NC),
        in_specs=[
            pl.BlockSpec((1, _NCH, _T, _H), lambda b, c: (b, c, 0, 0)),
            pl.BlockSpec((1, _T, _H), lambda b, c: (b, 0, 0)),
            pl.BlockSpec((_H, _H), lambda b, c: (0, 0)),
            pl.BlockSpec((1, _H), lambda b, c: (0, 0)),
            pl.BlockSpec((_H, _H), lambda b, c: (0, 0)),
            pl.BlockSpec((1, _H), lambda b, c: (0, 0)),
        ],
        out_specs=[
            pl.BlockSpec((1, _K, _T), lambda b, c: (b, 0, 0)),
            pl.BlockSpec((1, _K, _T), lambda b, c: (b, 0, 0)),
        ],
        out_shape=[
            jax.ShapeDtypeStruct((_B, _K, _T), jnp.float32),
            jax.ShapeDtypeStruct((_B, _K, _T), jnp.int32),
        ],
        scratch_shapes=[
            pltpu.VMEM((_H, _T), jnp.float32),
            pltpu.VMEM((_H, _T), jnp.float32),
            pltpu.VMEM((_H, _T), jnp.float32),
            pltpu.VMEM((_H, _NCH * _T), jnp.float32),
            pltpu.VMEM((_CPAD, _T), jnp.float32),
        ],
    )(peer_t, tgt_t, Wq, bq_row, Wk, bk_row)


_WINDOW = 128


def _gather_rows(peer_flat, idx_flat):
    num = _B * _K * _T
    mesh = plsc.VectorSubcoreMesh(core_axis_name="core", subcore_axis_name="subcore")

    wins_per_row = _T // _WINDOW

    @pl.kernel(out_type=jax.ShapeDtypeStruct((num, _H), jnp.float32), mesh=mesh)
    def sc_kernel(x_hbm, i_hbm, o_hbm):
        def body(i_vmem, o_vmem):
            pltpu.sync_copy(x_hbm.at[i_vmem.at[0]], o_vmem)

        pltpu.emit_pipeline(
            body,
            grid=(num // _WINDOW,),
            in_specs=[pl.BlockSpec(
                (1, _WINDOW),
                lambda i: (i // wins_per_row, i % wins_per_row))],
            out_specs=[pl.BlockSpec((_WINDOW, _H), lambda i: (i, 0))],
            core_axis_name=("core", "subcore"),
            dimension_semantics=(pltpu.PARALLEL,),
        )(i_hbm, o_hbm)

    return sc_kernel(peer_flat, idx_flat)


def _combine_body(v_ref, w_ref, wv_ref, bv_ref, w1_ref, b1_ref, w2_ref, b2_ref,
                  g_ref, be_ref, out_ref):
    v8 = jax.lax.dot_general(v_ref[0], wv_ref[...], (((1,), (1,)), ((), ())),
                             preferred_element_type=jnp.float32) + bv_ref[...]
    w = w_ref[0]
    wT = jnp.transpose(w)
    ll = wT[:, 0:1] * v8[0:_T, :]
    for j in range(1, _K):
        ll = ll + wT[:, j:j + 1] * v8[j * _T:(j + 1) * _T, :]
    h1 = jax.lax.dot_general(ll, w1_ref[...], (((1,), (1,)), ((), ())),
                             preferred_element_type=jnp.float32) + b1_ref[...]
    h1 = jnp.where(h1 > 0, h1, jnp.exp(jnp.minimum(h1, 0.0)) - 1.0)
    y = ll + jax.lax.dot_general(h1, w2_ref[...], (((1,), (1,)), ((), ())),
                                 preferred_element_type=jnp.float32) + b2_ref[...]
    mu = jnp.mean(y, axis=1, keepdims=True)
    d = y - mu
    var = jnp.mean(d * d, axis=1, keepdims=True)
    out_ref[0] = d * jax.lax.rsqrt(var + 1e-5) * g_ref[...] + be_ref[...]


def _combine(v, w, Wv, bv_row, W1, b1_row, W2, b2_row, g_row, be_row):
    full = lambda b: (0, 0)
    return pl.pallas_call(
        _combine_body,
        grid=(_B,),
        in_specs=[
            pl.BlockSpec((1, _K * _T, _H), lambda b: (b, 0, 0)),
            pl.BlockSpec((1, _K, _T), lambda b: (b, 0, 0)),
            pl.BlockSpec((_H, _H), full),
            pl.BlockSpec((1, _H), full),
            pl.BlockSpec((_H, _H), full),
            pl.BlockSpec((1, _H), full),
            pl.BlockSpec((_H, _H), full),
            pl.BlockSpec((1, _H), full),
            pl.BlockSpec((1, _H), full),
            pl.BlockSpec((1, _H), full),
        ],
        out_specs=pl.BlockSpec((1, _T, _H), lambda b: (b, 0, 0)),
        out_shape=jax.ShapeDtypeStruct((_B, _T, _H), jnp.float32),
    )(v, w, Wv, bv_row, W1, b1_row, W2, b2_row, g_row, be_row)


def kernel(target_h, peer_h, Wq, bq, Wk, bk, Wv, bv, W1, b1, W2, b2, gamma, beta):
    w, fidx = _select(peer_h, target_h, Wq, bq.reshape(1, _H), Wk, bk.reshape(1, _H))
    v = _gather_rows(peer_h.reshape(_B * _N * _T, _H),
                     fidx.reshape(_B * _K, _T))
    return _combine(v.reshape(_B, _K * _T, _H), w, Wv, bv.reshape(1, _H),
                    W1, b1.reshape(1, _H), W2, b2.reshape(1, _H),
                    gamma.reshape(1, _H), beta.reshape(1, _H))


# speedup vs baseline: 1.0509x; 1.0509x over previous
import jax
import jax.numpy as jnp
from jax.experimental import pallas as pl
from jax.experimental.pallas import tpu as pltpu
from jax.experimental.pallas import tpu_sc as plsc

_LAGS = (1, 5, 21)
_K = 8
_B, _N, _T, _H = 2, 32, 512, 256
_L = len(_LAGS)
_CPAD = 96
_NEG_INF = float("-inf")
_SCALE = 0.0625


_NC = 4
_NCH = _N // _NC


def _select_body(peer_ref, tgt_ref, wq_ref, bq_ref, wk_ref, bk_ref,
                 w_ref, idx_ref, qk1_ref, qk5_ref, qk21_ref, kb_ref, logits_ref):
    b = pl.program_id(0)
    c = pl.program_id(1)

    @pl.when(c == 0)
    def _():
        qT = jax.lax.dot_general(wq_ref[...], tgt_ref[0], (((1,), (1,)), ((), ())),
                                 preferred_element_type=jnp.float32)
        qT = ((qT + jnp.transpose(bq_ref[...])).astype(jnp.bfloat16)
              .astype(jnp.float32) * _SCALE).astype(jnp.bfloat16)
        for lag, qref in ((1, qk1_ref), (5, qk5_ref), (21, qk21_ref)):
            qref[:, : _T - lag] = qT[:, lag:]
            qref[:, _T - lag:] = jnp.zeros((_H, lag), jnp.bfloat16)

    kT = jax.lax.dot_general(
        wk_ref[...], peer_ref[0].reshape(_NCH * _T, _H), (((1,), (1,)), ((), ())),
        preferred_element_type=jnp.float32)
    kb_ref[...] = (kT + jnp.transpose(bk_ref[...])).astype(jnp.bfloat16)

    def n_body(i, carry):
        for j in (i, i + _NCH // 2):
            kb = kb_ref[:, pl.ds(j * _T, _T)].astype(jnp.float32)
            n = c * _NCH + j
            for l, (lag, qref) in enumerate(((1, qk1_ref), (5, qk5_ref), (21, qk21_ref))):
                qs = qref[...].astype(jnp.float32)
                d = jnp.sum(qs * kb, axis=0, keepdims=True)
                logits_ref[pl.ds(l * _N + n, 1), pl.ds(lag, _T - lag)] = d[:, : _T - lag]
        return carry

    jax.lax.fori_loop(0, _NCH // 2, n_body, 0)

    @pl.when(c == _NC - 1)
    def _():
        _finish(b, w_ref, idx_ref, logits_ref)


def _finish(b, w_ref, idx_ref, logits_ref):
    for l, lag in enumerate(_LAGS):
        logits_ref[pl.ds(l * _N, _N), pl.ds(0, lag)] = jnp.full((_N, lag), _NEG_INF,
                                                                jnp.float32)

    arr = logits_ref[...]
    rows = jax.lax.broadcasted_iota(jnp.int32, (_CPAD, _T), 0)
    vals, idxs = [], []
    for _ in range(_K):
        mmax = jnp.max(arr, axis=0, keepdims=True)
        sel = jnp.where(arr == mmax, rows, _CPAD)
        amin = jnp.min(sel, axis=0, keepdims=True)
        vals.append(mmax)
        idxs.append(amin)
        arr = jnp.where(rows == amin, _NEG_INF, arr)
    vals8 = jnp.concatenate(vals, axis=0)
    idx8 = jnp.concatenate(idxs, axis=0)

    allinf = vals8[0:1, :] == _NEG_INF
    safe = jnp.maximum(vals8, -1e9)
    e = jnp.exp(safe - safe[0:1, :])
    w = e / jnp.sum(e, axis=0, keepdims=True)
    w = jnp.where(allinf, 0.0, w)

    l_id = jax.lax.shift_right_logical(idx8, 5)
    n_id = jnp.bitwise_and(idx8, 31)
    lag_v = jnp.where(l_id == 0, 1, jnp.where(l_id == 1, 5, 21))
    t_iota = jax.lax.broadcasted_iota(jnp.int32, (_K, _T), 1)
    tt = jnp.maximum(t_iota - lag_v, 0)
    w_ref[0] = w
    idx_ref[0] = (b * _N + n_id) * _T + tt


def _select(peer_t, tgt_t, Wq, bq_row, Wk, bk_row):
    return pl.pallas_call(
        _select_body,
        grid=(_B, _NC),
        in_specs=[
            pl.BlockSpec((1, _NCH, _T, _H), lambda b, c: (b, c, 0, 0)),
            pl.BlockSpec((1, _T, _H), lambda b, c: (b, 0, 0)),
            pl.BlockSpec((_H, _H), lambda b, c: (0, 0)),
            pl.BlockSpec((1, _H), lambda b, c: (0, 0)),
            pl.BlockSpec((_H, _H), lambda b, c: (0, 0)),
            pl.BlockSpec((1, _H), lambda b, c: (0, 0)),
        ],
        out_specs=[
            pl.BlockSpec((1, _K, _T), lambda b, c: (b, 0, 0)),
            pl.BlockSpec((1, _K, _T), lambda b, c: (b, 0, 0)),
        ],
        out_shape=[
            jax.ShapeDtypeStruct((_B, _K, _T), jnp.float32),
            jax.ShapeDtypeStruct((_B, _K, _T), jnp.int32),
        ],
        scratch_shapes=[
            pltpu.VMEM((_H, _T), jnp.bfloat16),
            pltpu.VMEM((_H, _T), jnp.bfloat16),
            pltpu.VMEM((_H, _T), jnp.bfloat16),
            pltpu.VMEM((_H, _NCH * _T), jnp.bfloat16),
            pltpu.VMEM((_CPAD, _T), jnp.float32),
        ],
    )(peer_t, tgt_t, Wq, bq_row, Wk, bk_row)


_WINDOW = 128


def _gather_rows(peer_flat, idx_flat):
    num = _B * _K * _T
    mesh = plsc.VectorSubcoreMesh(core_axis_name="core", subcore_axis_name="subcore")

    wins_per_row = _T // _WINDOW

    @pl.kernel(out_type=jax.ShapeDtypeStruct((num, _H), jnp.float32), mesh=mesh)
    def sc_kernel(x_hbm, i_hbm, o_hbm):
        def body(i_vmem, o_vmem):
            pltpu.sync_copy(x_hbm.at[i_vmem.at[0]], o_vmem)

        pltpu.emit_pipeline(
            body,
            grid=(num // _WINDOW,),
            in_specs=[pl.BlockSpec(
                (1, _WINDOW),
                lambda i: (i // wins_per_row, i % wins_per_row))],
            out_specs=[pl.BlockSpec((_WINDOW, _H), lambda i: (i, 0))],
            core_axis_name=("core", "subcore"),
            dimension_semantics=(pltpu.PARALLEL,),
        )(i_hbm, o_hbm)

    return sc_kernel(peer_flat, idx_flat)


def _combine_body(v_ref, w_ref, wv_ref, bv_ref, w1_ref, b1_ref, w2_ref, b2_ref,
                  g_ref, be_ref, out_ref):
    v8 = jax.lax.dot_general(v_ref[0], wv_ref[...], (((1,), (1,)), ((), ())),
                             preferred_element_type=jnp.float32) + bv_ref[...]
    w = w_ref[0]
    wT = jnp.transpose(w)
    ll = wT[:, 0:1] * v8[0:_T, :]
    for j in range(1, _K):
        ll = ll + wT[:, j:j + 1] * v8[j * _T:(j + 1) * _T, :]
    h1 = jax.lax.dot_general(ll, w1_ref[...], (((1,), (1,)), ((), ())),
                             preferred_element_type=jnp.float32) + b1_ref[...]
    h1 = jnp.where(h1 > 0, h1, jnp.exp(jnp.minimum(h1, 0.0)) - 1.0)
    y = ll + jax.lax.dot_general(h1, w2_ref[...], (((1,), (1,)), ((), ())),
                                 preferred_element_type=jnp.float32) + b2_ref[...]
    mu = jnp.mean(y, axis=1, keepdims=True)
    d = y - mu
    var = jnp.mean(d * d, axis=1, keepdims=True)
    out_ref[0] = d * jax.lax.rsqrt(var + 1e-5) * g_ref[...] + be_ref[...]


def _combine(v, w, Wv, bv_row, W1, b1_row, W2, b2_row, g_row, be_row):
    full = lambda b: (0, 0)
    return pl.pallas_call(
        _combine_body,
        grid=(_B,),
        in_specs=[
            pl.BlockSpec((1, _K * _T, _H), lambda b: (b, 0, 0)),
            pl.BlockSpec((1, _K, _T), lambda b: (b, 0, 0)),
            pl.BlockSpec((_H, _H), full),
            pl.BlockSpec((1, _H), full),
            pl.BlockSpec((_H, _H), full),
            pl.BlockSpec((1, _H), full),
            pl.BlockSpec((_H, _H), full),
            pl.BlockSpec((1, _H), full),
            pl.BlockSpec((1, _H), full),
            pl.BlockSpec((1, _H), full),
        ],
        out_specs=pl.BlockSpec((1, _T, _H), lambda b: (b, 0, 0)),
        out_shape=jax.ShapeDtypeStruct((_B, _T, _H), jnp.float32),
    )(v, w, Wv, bv_row, W1, b1_row, W2, b2_row, g_row, be_row)


def kernel(target_h, peer_h, Wq, bq, Wk, bk, Wv, bv, W1, b1, W2, b2, gamma, beta):
    w, fidx = _select(peer_h, target_h, Wq, bq.reshape(1, _H), Wk, bk.reshape(1, _H))
    v = _gather_rows(peer_h.reshape(_B * _N * _T, _H),
                     fidx.reshape(_B * _K, _T))
    return _combine(v.reshape(_B, _K * _T, _H), w, Wv, bv.reshape(1, _H),
                    W1, b1.reshape(1, _H), W2, b2.reshape(1, _H),
                    gamma.reshape(1, _H), beta.reshape(1, _H))
